# Initial kernel scaffold; baseline (speedup 1.0000x reference)
import functools

import jax
import jax.numpy as jnp
from jax import lax
from jax.experimental import pallas as pl
from jax.experimental.pallas import tpu as pltpu
from jax.experimental.pallas import tpu_sc as plsc

NC = 2
NS = 16
NW = NC * NS
C = 128


def _make_sc_scatter(n_agg, d, k_chunks):
    rows_per_sub = n_agg // NS
    mesh = plsc.VectorSubcoreMesh(
        core_axis_name="c", subcore_axis_name="s", num_cores=NC, num_subcores=NS
    )

    @functools.partial(
        pl.kernel,
        out_type=jax.ShapeDtypeStruct((NC, n_agg, d), jnp.float32),
        mesh=mesh,
        scratch_types=[
            pltpu.VMEM((k_chunks, C), jnp.int32),
            pltpu.VMEM((k_chunks, C), jnp.int32),
            pltpu.VMEM((C, d), jnp.float32),
            pltpu.VMEM_SHARED((n_agg, d), jnp.float32),
            pltpu.SemaphoreType.DMA,
            pltpu.SemaphoreType.DMA,
        ],
    )
    def sc_scatter(u_hbm, src_hbm, dst_hbm, zeros_hbm, out_hbm,
                   src_v, dst_v, rows_v, agg_sh, sem_g, sem_s):
        c = lax.axis_index("c")
        s = lax.axis_index("s")
        wid = c * NS + s
        sub_rows = pl.ds(s * rows_per_sub, rows_per_sub)
        pltpu.sync_copy(zeros_hbm.at[sub_rows], agg_sh.at[sub_rows])
        pltpu.sync_copy(src_hbm.at[pl.ds(wid * k_chunks, k_chunks)], src_v)
        pltpu.sync_copy(dst_hbm.at[pl.ds(wid * k_chunks, k_chunks)], dst_v)
        plsc.subcore_barrier()

        def body(j, carry):
            pltpu.async_copy(u_hbm.at[src_v.at[j]], rows_v, sem_g).wait()
            pltpu.async_copy(rows_v, agg_sh.at[dst_v.at[j]], sem_s,
                             add=True).wait()
            return carry

        lax.fori_loop(0, k_chunks, body, 0)
        plsc.subcore_barrier()
        pltpu.sync_copy(agg_sh.at[sub_rows], out_hbm.at[c, sub_rows])

    return sc_scatter


def _matmul_t(a, w):
    return lax.dot_general(a, w, (((1,), (1,)), ((), ())),
                           preferred_element_type=jnp.float32)


def _xlin_body(x_ref, w_ref, b_ref, out_ref):
    out_ref[...] = _matmul_t(x_ref[...], w_ref[...]) + b_ref[...]


def _mlp_body(agg0_ref, agg1_ref, xlin_ref, s1w, s1b, s2w, s2b, s3w, s3b,
              out_ref):
    a = agg0_ref[0] + agg1_ref[0]
    h = jax.nn.relu(_matmul_t(a, s1w[...]) + s1b[...])
    h = jax.nn.relu(_matmul_t(h, s2w[...]) + s2b[...])
    h = jnp.tanh(_matmul_t(h, s3w[...]) + s3b[...])
    out_ref[...] = jax.nn.relu(xlin_ref[...] + h)


def _readout_body(u_ref, w2_ref, b2_ref, out_ref, acc_ref):
    i = pl.program_id(0)

    @pl.when(i == 0)
    def _():
        acc_ref[...] = jnp.zeros_like(acc_ref)

    acc_ref[...] += jnp.sum(u_ref[...], axis=0, keepdims=True)

    @pl.when(i == pl.num_programs(0) - 1)
    def _():
        out_ref[...] = _matmul_t(acc_ref[...], w2_ref[...]) + b2_ref[...]


def kernel(x, u, edge_index, w1_w, w1_b, s1_w, s1_b, s2_w, s2_b, s3_w, s3_b,
           w2_w, w2_b):
    n, d = u.shape
    t_rounds = 4

    n_agg = ((n + 2 * NS - 1) // (2 * NS)) * 2 * NS + NS * 16
    e = edge_index.shape[1]
    k_chunks = (e + NW * C - 1) // (NW * C)
    e_pad = NW * k_chunks * C
    n_trash = n_agg - n

    pad = e_pad - e
    pad_ar = jnp.arange(pad, dtype=jnp.int32)
    src_p = jnp.concatenate([edge_index[0], pad_ar % n]).reshape(NW * k_chunks, C)
    dst_p = jnp.concatenate([edge_index[1], n + pad_ar % n_trash]).reshape(
        NW * k_chunks, C)
    zeros_hbm = jnp.zeros((n_agg, d), jnp.float32)

    sc_scatter = _make_sc_scatter(n_agg, d, k_chunks)

    bn = 2000
    grid = (n // bn,)
    w_spec = pl.BlockSpec((d, d), lambda i: (0, 0))
    b_spec = pl.BlockSpec((1, d), lambda i: (0, 0))
    row_spec = pl.BlockSpec((bn, d), lambda i: (i, 0))

    xlin = pl.pallas_call(
        _xlin_body,
        grid=grid,
        in_specs=[row_spec, w_spec, b_spec],
        out_specs=row_spec,
        out_shape=jax.ShapeDtypeStruct((n, d), jnp.float32),
    )(x, w1_w, w1_b.reshape(1, d))

    mlp = pl.pallas_call(
        _mlp_body,
        grid=grid,
        in_specs=[
            pl.BlockSpec((1, bn, d), lambda i: (0, i, 0)),
            pl.BlockSpec((1, bn, d), lambda i: (1, i, 0)),
            row_spec, w_spec, b_spec, w_spec, b_spec, w_spec, b_spec,
        ],
        out_specs=row_spec,
        out_shape=jax.ShapeDtypeStruct((n, d), jnp.float32),
    )

    s1b2, s2b2, s3b2 = (b.reshape(1, d) for b in (s1_b, s2_b, s3_b))
    for _ in range(t_rounds):
        agg2 = sc_scatter(u, src_p, dst_p, zeros_hbm)
        u = mlp(agg2, agg2, xlin, s1_w, s1b2, s2_w, s2b2, s3_w, s3b2)

    g = pl.pallas_call(
        _readout_body,
        grid=grid,
        in_specs=[row_spec, w_spec, b_spec],
        out_specs=pl.BlockSpec((1, d), lambda i: (0, 0)),
        out_shape=jax.ShapeDtypeStruct((1, d), jnp.float32),
        scratch_shapes=[pltpu.VMEM((1, d), jnp.float32)],
    )(u, w2_w, w2_b.reshape(1, d))
    return g


# speedup vs baseline: 8.3231x; 8.3231x over previous
import functools

import jax
import jax.numpy as jnp
from jax import lax
from jax.experimental import pallas as pl
from jax.experimental.pallas import tpu as pltpu
from jax.experimental.pallas import tpu_sc as plsc

NC = 2
NS = 16
NW = NC * NS
C = 128


def _make_sc_scatter(n_agg, d, k_chunks):
    rows_per_sub = n_agg // NS
    mesh = plsc.VectorSubcoreMesh(
        core_axis_name="c", subcore_axis_name="s", num_cores=NC, num_subcores=NS
    )

    @functools.partial(
        pl.kernel,
        out_type=jax.ShapeDtypeStruct((NC, n_agg, d), jnp.float32),
        mesh=mesh,
        scratch_types=[
            pltpu.VMEM((k_chunks, C), jnp.int32),
            pltpu.VMEM((k_chunks, C), jnp.int32),
            pltpu.VMEM((C, d), jnp.float32),
            pltpu.VMEM_SHARED((n_agg, d), jnp.float32),
            pltpu.SemaphoreType.DMA,
            pltpu.SemaphoreType.DMA,
        ],
    )
    def sc_scatter(u_hbm, src_hbm, dst_hbm, zeros_hbm, out_hbm,
                   src_v, dst_v, rows_v, agg_sh, sem_g, sem_s):
        c = lax.axis_index("c")
        s = lax.axis_index("s")
        wid = c * NS + s
        sub_rows = pl.ds(s * rows_per_sub, rows_per_sub)
        pltpu.sync_copy(zeros_hbm.at[sub_rows], agg_sh.at[sub_rows])
        pltpu.sync_copy(src_hbm.at[pl.ds(wid * k_chunks, k_chunks)], src_v)
        pltpu.sync_copy(dst_hbm.at[pl.ds(wid * k_chunks, k_chunks)], dst_v)
        plsc.subcore_barrier()

        def body(j, carry):
            pltpu.async_copy(u_hbm.at[src_v.at[j]], rows_v, sem_g).wait()
            pltpu.async_copy(rows_v, agg_sh.at[dst_v.at[j]], sem_s,
                             add=True).wait()
            return carry

        lax.fori_loop(0, k_chunks, body, 0)
        plsc.subcore_barrier()
        pltpu.sync_copy(agg_sh.at[sub_rows], out_hbm.at[c, sub_rows])

    return sc_scatter


def _matmul_t(a, w):
    return lax.dot_general(a, w, (((1,), (1,)), ((), ())),
                           preferred_element_type=jnp.float32)


def _xlin_body(x_ref, w_ref, b_ref, out_ref):
    out_ref[...] = _matmul_t(x_ref[...], w_ref[...]) + b_ref[...]


def _mlp_body(agg0_ref, agg1_ref, xlin_ref, s1w, s1b, s2w, s2b, s3w, s3b,
              out_ref):
    a = agg0_ref[0] + agg1_ref[0]
    h = jax.nn.relu(_matmul_t(a, s1w[...]) + s1b[...])
    h = jax.nn.relu(_matmul_t(h, s2w[...]) + s2b[...])
    h = jnp.tanh(_matmul_t(h, s3w[...]) + s3b[...])
    out_ref[...] = jax.nn.relu(xlin_ref[...] + h)


def _readout_body(u_ref, w2_ref, b2_ref, out_ref, acc_ref):
    i = pl.program_id(0)

    @pl.when(i == 0)
    def _():
        acc_ref[...] = jnp.zeros_like(acc_ref)

    acc_ref[...] += jnp.sum(u_ref[...], axis=0, keepdims=True)

    @pl.when(i == pl.num_programs(0) - 1)
    def _():
        out_ref[...] = _matmul_t(acc_ref[...], w2_ref[...]) + b2_ref[...]


def kernel(x, u, edge_index, w1_w, w1_b, s1_w, s1_b, s2_w, s2_b, s3_w, s3_b,
           w2_w, w2_b):
    n, d = u.shape
    t_rounds = 4

    n_agg = ((n + 8 * NS - 1) // (8 * NS)) * 8 * NS + 8 * NS
    e = edge_index.shape[1]
    k_chunks = ((e + NW * C - 1) // (NW * C) + 7) // 8 * 8
    e_pad = NW * k_chunks * C
    n_trash = n_agg - n

    pad = e_pad - e
    pad_ar = jnp.arange(pad, dtype=jnp.int32)
    src_p = jnp.concatenate([edge_index[0], pad_ar % n]).reshape(NW * k_chunks, C)
    dst_p = jnp.concatenate([edge_index[1], n + pad_ar % n_trash]).reshape(
        NW * k_chunks, C)
    zeros_hbm = jnp.zeros((n_agg, d), jnp.float32)

    sc_scatter = _make_sc_scatter(n_agg, d, k_chunks)

    bn = 2000
    grid = (n // bn,)
    w_spec = pl.BlockSpec((d, d), lambda i: (0, 0))
    b_spec = pl.BlockSpec((1, d), lambda i: (0, 0))
    row_spec = pl.BlockSpec((bn, d), lambda i: (i, 0))

    xlin = pl.pallas_call(
        _xlin_body,
        grid=grid,
        in_specs=[row_spec, w_spec, b_spec],
        out_specs=row_spec,
        out_shape=jax.ShapeDtypeStruct((n, d), jnp.float32),
    )(x, w1_w, w1_b.reshape(1, d))

    mlp = pl.pallas_call(
        _mlp_body,
        grid=grid,
        in_specs=[
            pl.BlockSpec((1, bn, d), lambda i: (0, i, 0)),
            pl.BlockSpec((1, bn, d), lambda i: (1, i, 0)),
            row_spec, w_spec, b_spec, w_spec, b_spec, w_spec, b_spec,
        ],
        out_specs=row_spec,
        out_shape=jax.ShapeDtypeStruct((n, d), jnp.float32),
    )

    s1b2, s2b2, s3b2 = (b.reshape(1, d) for b in (s1_b, s2_b, s3_b))
    for _ in range(t_rounds):
        agg2 = sc_scatter(u, src_p, dst_p, zeros_hbm)
        u = mlp(agg2, agg2, xlin, s1_w, s1b2, s2_w, s2b2, s3_w, s3b2)

    g = pl.pallas_call(
        _readout_body,
        grid=grid,
        in_specs=[row_spec, w_spec, b_spec],
        out_specs=pl.BlockSpec((1, d), lambda i: (0, 0)),
        out_shape=jax.ShapeDtypeStruct((1, d), jnp.float32),
        scratch_shapes=[pltpu.VMEM((1, d), jnp.float32)],
    )(u, w2_w, w2_b.reshape(1, d))
    return g


# speedup vs baseline: 10.7308x; 1.2893x over previous
import functools

import jax
import jax.numpy as jnp
from jax import lax
from jax.experimental import pallas as pl
from jax.experimental.pallas import tpu as pltpu
from jax.experimental.pallas import tpu_sc as plsc

NC = 2
NS = 16
NW = NC * NS
C = 128


def _make_sc_scatter(n_agg, d, k_chunks):
    rows_per_sub = n_agg // NS
    kh = k_chunks // 2
    mesh = plsc.VectorSubcoreMesh(
        core_axis_name="c", subcore_axis_name="s", num_cores=NC, num_subcores=NS
    )

    @functools.partial(
        pl.kernel,
        out_type=jax.ShapeDtypeStruct((NC, n_agg, d), jnp.float32),
        mesh=mesh,
        scratch_types=[
            pltpu.VMEM((kh, C), jnp.int32),
            pltpu.VMEM((kh, C), jnp.int32),
            pltpu.VMEM((C, d), jnp.float32),
            pltpu.VMEM((C, d), jnp.float32),
            pltpu.VMEM_SHARED((n_agg, d), jnp.float32),
            pltpu.SemaphoreType.DMA,
            pltpu.SemaphoreType.DMA,
        ],
    )
    def sc_scatter(u_hbm, src_hbm, dst_hbm, zeros_hbm, out_hbm,
                   src_v, dst_v, rows_a, rows_b, agg_sh, sem_g, sem_s):
        c = lax.axis_index("c")
        s = lax.axis_index("s")
        wid = c * NS + s
        sub_rows = pl.ds(s * rows_per_sub, rows_per_sub)
        pltpu.sync_copy(zeros_hbm.at[sub_rows], agg_sh.at[sub_rows])
        barrier_done = False

        for h in range(2):
            pltpu.sync_copy(
                src_hbm.at[pl.ds(wid * k_chunks + h * kh, kh)], src_v)
            pltpu.sync_copy(
                dst_hbm.at[pl.ds(wid * k_chunks + h * kh, kh)], dst_v)
            if not barrier_done:
                plsc.subcore_barrier()
                barrier_done = True

            n_pairs = kh // 2
            pltpu.async_copy(u_hbm.at[src_v.at[0]], rows_a, sem_g)

            def body(i, carry):
                je = 2 * i
                pltpu.make_async_copy(u_hbm.at[src_v.at[je]], rows_a,
                                      sem_g).wait()

                @pl.when(i >= 1)
                def _():
                    pltpu.make_async_copy(rows_b, agg_sh.at[dst_v.at[je - 1]],
                                          sem_s).wait()

                pltpu.async_copy(u_hbm.at[src_v.at[je + 1]], rows_b, sem_g)
                pltpu.async_copy(rows_a, agg_sh.at[dst_v.at[je]], sem_s,
                                 add=True)
                pltpu.make_async_copy(u_hbm.at[src_v.at[je + 1]], rows_b,
                                      sem_g).wait()
                pltpu.make_async_copy(rows_a, agg_sh.at[dst_v.at[je]],
                                      sem_s).wait()

                @pl.when(i + 1 < n_pairs)
                def _():
                    pltpu.async_copy(u_hbm.at[src_v.at[je + 2]], rows_a,
                                     sem_g)

                pltpu.async_copy(rows_b, agg_sh.at[dst_v.at[je + 1]], sem_s,
                                 add=True)
                return carry

            lax.fori_loop(0, n_pairs, body, 0)
            pltpu.make_async_copy(rows_b, agg_sh.at[dst_v.at[kh - 1]],
                                  sem_s).wait()

        plsc.subcore_barrier()
        pltpu.sync_copy(agg_sh.at[sub_rows], out_hbm.at[c, sub_rows])

    return sc_scatter


def _matmul_t(a, w):
    return lax.dot_general(a, w, (((1,), (1,)), ((), ())),
                           preferred_element_type=jnp.float32)


def _xlin_body(x_ref, w_ref, b_ref, out_ref):
    out_ref[...] = _matmul_t(x_ref[...], w_ref[...]) + b_ref[...]


def _mlp_body(agg0_ref, agg1_ref, xlin_ref, s1w, s1b, s2w, s2b, s3w, s3b,
              out_ref):
    a = agg0_ref[0] + agg1_ref[0]
    h = jax.nn.relu(_matmul_t(a, s1w[...]) + s1b[...])
    h = jax.nn.relu(_matmul_t(h, s2w[...]) + s2b[...])
    h = jnp.tanh(_matmul_t(h, s3w[...]) + s3b[...])
    out_ref[...] = jax.nn.relu(xlin_ref[...] + h)


def _readout_body(u_ref, w2_ref, b2_ref, out_ref, acc_ref):
    i = pl.program_id(0)

    @pl.when(i == 0)
    def _():
        acc_ref[...] = jnp.zeros_like(acc_ref)

    acc_ref[...] += jnp.sum(u_ref[...], axis=0, keepdims=True)

    @pl.when(i == pl.num_programs(0) - 1)
    def _():
        out_ref[...] = _matmul_t(acc_ref[...], w2_ref[...]) + b2_ref[...]


def kernel(x, u, edge_index, w1_w, w1_b, s1_w, s1_b, s2_w, s2_b, s3_w, s3_b,
           w2_w, w2_b):
    n, d = u.shape
    t_rounds = 4

    n_agg = ((n + 8 * NS - 1) // (8 * NS)) * 8 * NS
    if n_agg == n:
        n_agg += 8 * NS
    e = edge_index.shape[1]
    k_chunks = ((e + NW * C - 1) // (NW * C) + 15) // 16 * 16
    e_pad = NW * k_chunks * C
    n_trash = n_agg - n

    pad = e_pad - e
    pad_ar = jnp.arange(pad, dtype=jnp.int32)
    src_p = jnp.concatenate([edge_index[0], pad_ar % n]).reshape(NW * k_chunks, C)
    dst_p = jnp.concatenate([edge_index[1], n + pad_ar % n_trash]).reshape(
        NW * k_chunks, C)
    zeros_hbm = jnp.zeros((n_agg, d), jnp.float32)

    sc_scatter = _make_sc_scatter(n_agg, d, k_chunks)

    bn = 2000
    grid = (n // bn,)
    w_spec = pl.BlockSpec((d, d), lambda i: (0, 0))
    b_spec = pl.BlockSpec((1, d), lambda i: (0, 0))
    row_spec = pl.BlockSpec((bn, d), lambda i: (i, 0))

    xlin = pl.pallas_call(
        _xlin_body,
        grid=grid,
        in_specs=[row_spec, w_spec, b_spec],
        out_specs=row_spec,
        out_shape=jax.ShapeDtypeStruct((n, d), jnp.float32),
    )(x, w1_w, w1_b.reshape(1, d))

    mlp = pl.pallas_call(
        _mlp_body,
        grid=grid,
        in_specs=[
            pl.BlockSpec((1, bn, d), lambda i: (0, i, 0)),
            pl.BlockSpec((1, bn, d), lambda i: (1, i, 0)),
            row_spec, w_spec, b_spec, w_spec, b_spec, w_spec, b_spec,
        ],
        out_specs=row_spec,
        out_shape=jax.ShapeDtypeStruct((n, d), jnp.float32),
    )

    s1b2, s2b2, s3b2 = (b.reshape(1, d) for b in (s1_b, s2_b, s3_b))
    for _ in range(t_rounds):
        agg2 = sc_scatter(u, src_p, dst_p, zeros_hbm)
        u = mlp(agg2, agg2, xlin, s1_w, s1b2, s2_w, s2b2, s3_w, s3b2)

    g = pl.pallas_call(
        _readout_body,
        grid=grid,
        in_specs=[row_spec, w_spec, b_spec],
        out_specs=pl.BlockSpec((1, d), lambda i: (0, 0)),
        out_shape=jax.ShapeDtypeStruct((1, d), jnp.float32),
        scratch_shapes=[pltpu.VMEM((1, d), jnp.float32)],
    )(u, w2_w, w2_b.reshape(1, d))
    return g


# speedup vs baseline: 11.0719x; 1.0318x over previous
import functools

import jax
import jax.numpy as jnp
from jax import lax
from jax.experimental import pallas as pl
from jax.experimental.pallas import tpu as pltpu
from jax.experimental.pallas import tpu_sc as plsc

NC = 2
NS = 16
NW = NC * NS
C = 64


def _make_sc_scatter(n_agg, d, k_chunks):
    rows_per_sub = n_agg // NS
    kh = k_chunks // 4
    mesh = plsc.VectorSubcoreMesh(
        core_axis_name="c", subcore_axis_name="s", num_cores=NC, num_subcores=NS
    )

    @functools.partial(
        pl.kernel,
        out_type=jax.ShapeDtypeStruct((NC, n_agg, d), jnp.float32),
        mesh=mesh,
        scratch_types=[
            pltpu.VMEM((kh, C), jnp.int32),
            pltpu.VMEM((kh, C), jnp.int32),
            pltpu.VMEM((4, C, d), jnp.float32),
            pltpu.VMEM_SHARED((n_agg, d), jnp.float32),
            pltpu.SemaphoreType.DMA,
            pltpu.SemaphoreType.DMA,
            pltpu.SemaphoreType.DMA,
            pltpu.SemaphoreType.DMA,
        ],
    )
    def sc_scatter(u_hbm, src_hbm, dst_hbm, zeros_hbm, out_hbm,
                   src_v, dst_v, rows4, agg_sh, g0, g1, s0, s1):
        c = lax.axis_index("c")
        s = lax.axis_index("s")
        wid = c * NS + s
        sub_rows = pl.ds(s * rows_per_sub, rows_per_sub)
        pltpu.sync_copy(zeros_hbm.at[sub_rows], agg_sh.at[sub_rows])
        barrier_done = False

        for h in range(4):
            pltpu.sync_copy(
                src_hbm.at[pl.ds(wid * k_chunks + h * kh, kh)], src_v)
            pltpu.sync_copy(
                dst_hbm.at[pl.ds(wid * k_chunks + h * kh, kh)], dst_v)
            if not barrier_done:
                plsc.subcore_barrier()
                barrier_done = True

            n_quads = kh // 4
            gsem = (g0, g1)
            ssem = (s0, s1)
            pltpu.async_copy(u_hbm.at[src_v.at[0]], rows4.at[0], g0)
            pltpu.async_copy(u_hbm.at[src_v.at[1]], rows4.at[1], g1)

            def body(q, carry):
                for b in range(4):
                    j = 4 * q + b
                    buf = rows4.at[b]
                    pltpu.make_async_copy(u_hbm.at[src_v.at[j]], buf,
                                          gsem[b % 2]).wait()

                    @pl.when(j >= 2)
                    def _():
                        pltpu.make_async_copy(
                            rows4.at[(b + 2) % 4],
                            agg_sh.at[dst_v.at[j - 2]], ssem[b % 2]).wait()

                    @pl.when(j + 2 < kh)
                    def _():
                        pltpu.async_copy(u_hbm.at[src_v.at[j + 2]],
                                         rows4.at[(b + 2) % 4], gsem[b % 2])

                    pltpu.async_copy(buf, agg_sh.at[dst_v.at[j]],
                                     ssem[b % 2], add=True)
                return carry

            lax.fori_loop(0, n_quads, body, 0)
            pltpu.make_async_copy(rows4.at[2], agg_sh.at[dst_v.at[kh - 2]],
                                  s0).wait()
            pltpu.make_async_copy(rows4.at[3], agg_sh.at[dst_v.at[kh - 1]],
                                  s1).wait()

        plsc.subcore_barrier()
        pltpu.sync_copy(agg_sh.at[sub_rows], out_hbm.at[c, sub_rows])

    return sc_scatter


def _matmul_t(a, w):
    return lax.dot_general(a, w, (((1,), (1,)), ((), ())),
                           preferred_element_type=jnp.float32)


def _xlin_body(x_ref, w_ref, b_ref, out_ref):
    out_ref[...] = _matmul_t(x_ref[...], w_ref[...]) + b_ref[...]


def _mlp_body(agg0_ref, agg1_ref, xlin_ref, s1w, s1b, s2w, s2b, s3w, s3b,
              out_ref):
    a = agg0_ref[0] + agg1_ref[0]
    h = jax.nn.relu(_matmul_t(a, s1w[...]) + s1b[...])
    h = jax.nn.relu(_matmul_t(h, s2w[...]) + s2b[...])
    h = jnp.tanh(_matmul_t(h, s3w[...]) + s3b[...])
    out_ref[...] = jax.nn.relu(xlin_ref[...] + h)


def _readout_body(u_ref, w2_ref, b2_ref, out_ref, acc_ref):
    i = pl.program_id(0)

    @pl.when(i == 0)
    def _():
        acc_ref[...] = jnp.zeros_like(acc_ref)

    acc_ref[...] += jnp.sum(u_ref[...], axis=0, keepdims=True)

    @pl.when(i == pl.num_programs(0) - 1)
    def _():
        out_ref[...] = _matmul_t(acc_ref[...], w2_ref[...]) + b2_ref[...]


def kernel(x, u, edge_index, w1_w, w1_b, s1_w, s1_b, s2_w, s2_b, s3_w, s3_b,
           w2_w, w2_b):
    n, d = u.shape
    t_rounds = 4

    n_agg = ((n + 8 * NS - 1) // (8 * NS)) * 8 * NS
    if n_agg == n:
        n_agg += 8 * NS
    e = edge_index.shape[1]
    k_chunks = ((e + NW * C - 1) // (NW * C) + 31) // 32 * 32
    e_pad = NW * k_chunks * C
    n_trash = n_agg - n

    pad = e_pad - e
    pad_ar = jnp.arange(pad, dtype=jnp.int32)
    src_p = jnp.concatenate([edge_index[0], pad_ar % n]).reshape(NW * k_chunks, C)
    dst_p = jnp.concatenate([edge_index[1], n + pad_ar % n_trash]).reshape(
        NW * k_chunks, C)
    zeros_hbm = jnp.zeros((n_agg, d), jnp.float32)

    sc_scatter = _make_sc_scatter(n_agg, d, k_chunks)

    bn = 2000
    grid = (n // bn,)
    w_spec = pl.BlockSpec((d, d), lambda i: (0, 0))
    b_spec = pl.BlockSpec((1, d), lambda i: (0, 0))
    row_spec = pl.BlockSpec((bn, d), lambda i: (i, 0))

    xlin = pl.pallas_call(
        _xlin_body,
        grid=grid,
        in_specs=[row_spec, w_spec, b_spec],
        out_specs=row_spec,
        out_shape=jax.ShapeDtypeStruct((n, d), jnp.float32),
    )(x, w1_w, w1_b.reshape(1, d))

    mlp = pl.pallas_call(
        _mlp_body,
        grid=grid,
        in_specs=[
            pl.BlockSpec((1, bn, d), lambda i: (0, i, 0)),
            pl.BlockSpec((1, bn, d), lambda i: (1, i, 0)),
            row_spec, w_spec, b_spec, w_spec, b_spec, w_spec, b_spec,
        ],
        out_specs=row_spec,
        out_shape=jax.ShapeDtypeStruct((n, d), jnp.float32),
    )

    s1b2, s2b2, s3b2 = (b.reshape(1, d) for b in (s1_b, s2_b, s3_b))
    for _ in range(t_rounds):
        agg2 = sc_scatter(u, src_p, dst_p, zeros_hbm)
        u = mlp(agg2, agg2, xlin, s1_w, s1b2, s2_w, s2b2, s3_w, s3b2)

    g = pl.pallas_call(
        _readout_body,
        grid=grid,
        in_specs=[row_spec, w_spec, b_spec],
        out_specs=pl.BlockSpec((1, d), lambda i: (0, 0)),
        out_shape=jax.ShapeDtypeStruct((1, d), jnp.float32),
        scratch_shapes=[pltpu.VMEM((1, d), jnp.float32)],
    )(u, w2_w, w2_b.reshape(1, d))
    return g


# speedup vs baseline: 12.1407x; 1.0965x over previous
import functools

import jax
import jax.numpy as jnp
from jax import lax
from jax.experimental import pallas as pl
from jax.experimental.pallas import tpu as pltpu
from jax.experimental.pallas import tpu_sc as plsc

NC = 2
NS = 16
NW = NC * NS
C = 32


def _make_sc_scatter(n_agg, d, k_chunks):
    rows_per_sub = n_agg // NS
    kh = k_chunks // 8
    mesh = plsc.VectorSubcoreMesh(
        core_axis_name="c", subcore_axis_name="s", num_cores=NC, num_subcores=NS
    )

    @functools.partial(
        pl.kernel,
        out_type=jax.ShapeDtypeStruct((NC, n_agg, d), jnp.float32),
        mesh=mesh,
        scratch_types=[
            pltpu.VMEM((kh, C), jnp.int32),
            pltpu.VMEM((kh, C), jnp.int32),
            pltpu.VMEM((8, C, d), jnp.float32),
            pltpu.VMEM_SHARED((n_agg, d), jnp.float32),
        ] + [pltpu.SemaphoreType.DMA] * 10,
    )
    def sc_scatter(u_hbm, src_hbm, dst_hbm, zeros_hbm, out_hbm,
                   src_v, dst_v, rows8, agg_sh, *sems):
        c = lax.axis_index("c")
        s = lax.axis_index("s")
        wid = c * NS + s
        sub_rows = pl.ds(s * rows_per_sub, rows_per_sub)
        pltpu.sync_copy(zeros_hbm.at[sub_rows], agg_sh.at[sub_rows])
        barrier_done = False

        gsem = sems[:8]
        ssem = sems[8:]
        for h in range(8):
            pltpu.sync_copy(
                src_hbm.at[pl.ds(wid * k_chunks + h * kh, kh)], src_v)
            pltpu.sync_copy(
                dst_hbm.at[pl.ds(wid * k_chunks + h * kh, kh)], dst_v)
            if not barrier_done:
                plsc.subcore_barrier()
                barrier_done = True

            n_octs = kh // 8
            for b in range(6):
                pltpu.async_copy(u_hbm.at[src_v.at[b]], rows8.at[b], gsem[b])

            def body(q, carry):
                for b in range(8):
                    j = 8 * q + b
                    buf = rows8.at[b]
                    pltpu.make_async_copy(u_hbm.at[src_v.at[j]], buf,
                                          gsem[b]).wait()

                    @pl.when(j >= 2)
                    def _():
                        pltpu.make_async_copy(
                            rows8.at[(b + 6) % 8],
                            agg_sh.at[dst_v.at[j - 2]], ssem[b % 2]).wait()

                    pltpu.async_copy(buf, agg_sh.at[dst_v.at[j]],
                                     ssem[b % 2], add=True)

                    @pl.when(j + 6 < kh)
                    def _():
                        pltpu.async_copy(u_hbm.at[src_v.at[j + 6]],
                                         rows8.at[(b + 6) % 8],
                                         gsem[(b + 6) % 8])
                return carry

            lax.fori_loop(0, n_octs, body, 0)
            pltpu.make_async_copy(rows8.at[(kh - 2) % 8],
                                  agg_sh.at[dst_v.at[kh - 2]], ssem[0]).wait()
            pltpu.make_async_copy(rows8.at[(kh - 1) % 8],
                                  agg_sh.at[dst_v.at[kh - 1]], ssem[1]).wait()

        plsc.subcore_barrier()
        pltpu.sync_copy(agg_sh.at[sub_rows], out_hbm.at[c, sub_rows])

    return sc_scatter


def _matmul_t(a, w):
    return lax.dot_general(a, w, (((1,), (1,)), ((), ())),
                           preferred_element_type=jnp.float32)


def _xlin_body(x_ref, w_ref, b_ref, out_ref):
    out_ref[...] = _matmul_t(x_ref[...], w_ref[...]) + b_ref[...]


def _mlp_body(agg0_ref, agg1_ref, xlin_ref, s1w, s1b, s2w, s2b, s3w, s3b,
              out_ref):
    a = agg0_ref[0] + agg1_ref[0]
    h = jax.nn.relu(_matmul_t(a, s1w[...]) + s1b[...])
    h = jax.nn.relu(_matmul_t(h, s2w[...]) + s2b[...])
    h = jnp.tanh(_matmul_t(h, s3w[...]) + s3b[...])
    out_ref[...] = jax.nn.relu(xlin_ref[...] + h)


def _readout_body(u_ref, w2_ref, b2_ref, out_ref, acc_ref):
    i = pl.program_id(0)

    @pl.when(i == 0)
    def _():
        acc_ref[...] = jnp.zeros_like(acc_ref)

    acc_ref[...] += jnp.sum(u_ref[...], axis=0, keepdims=True)

    @pl.when(i == pl.num_programs(0) - 1)
    def _():
        out_ref[...] = _matmul_t(acc_ref[...], w2_ref[...]) + b2_ref[...]


def kernel(x, u, edge_index, w1_w, w1_b, s1_w, s1_b, s2_w, s2_b, s3_w, s3_b,
           w2_w, w2_b):
    n, d = u.shape
    t_rounds = 4

    n_agg = ((n + 8 * NS - 1) // (8 * NS)) * 8 * NS
    if n_agg == n:
        n_agg += 8 * NS
    e = edge_index.shape[1]
    k_chunks = ((e + NW * C - 1) // (NW * C) + 63) // 64 * 64
    e_pad = NW * k_chunks * C
    n_trash = n_agg - n

    pad = e_pad - e
    pad_ar = jnp.arange(pad, dtype=jnp.int32)
    src_p = jnp.concatenate([edge_index[0], pad_ar % n]).reshape(NW * k_chunks, C)
    dst_p = jnp.concatenate([edge_index[1], n + pad_ar % n_trash]).reshape(
        NW * k_chunks, C)
    zeros_hbm = jnp.zeros((n_agg, d), jnp.float32)

    sc_scatter = _make_sc_scatter(n_agg, d, k_chunks)

    bn = 2000
    grid = (n // bn,)
    w_spec = pl.BlockSpec((d, d), lambda i: (0, 0))
    b_spec = pl.BlockSpec((1, d), lambda i: (0, 0))
    row_spec = pl.BlockSpec((bn, d), lambda i: (i, 0))

    xlin = pl.pallas_call(
        _xlin_body,
        grid=grid,
        in_specs=[row_spec, w_spec, b_spec],
        out_specs=row_spec,
        out_shape=jax.ShapeDtypeStruct((n, d), jnp.float32),
    )(x, w1_w, w1_b.reshape(1, d))

    mlp = pl.pallas_call(
        _mlp_body,
        grid=grid,
        in_specs=[
            pl.BlockSpec((1, bn, d), lambda i: (0, i, 0)),
            pl.BlockSpec((1, bn, d), lambda i: (1, i, 0)),
            row_spec, w_spec, b_spec, w_spec, b_spec, w_spec, b_spec,
        ],
        out_specs=row_spec,
        out_shape=jax.ShapeDtypeStruct((n, d), jnp.float32),
    )

    s1b2, s2b2, s3b2 = (b.reshape(1, d) for b in (s1_b, s2_b, s3_b))
    for _ in range(t_rounds):
        agg2 = sc_scatter(u, src_p, dst_p, zeros_hbm)
        u = mlp(agg2, agg2, xlin, s1_w, s1b2, s2_w, s2b2, s3_w, s3b2)

    g = pl.pallas_call(
        _readout_body,
        grid=grid,
        in_specs=[row_spec, w_spec, b_spec],
        out_specs=pl.BlockSpec((1, d), lambda i: (0, 0)),
        out_shape=jax.ShapeDtypeStruct((1, d), jnp.float32),
        scratch_shapes=[pltpu.VMEM((1, d), jnp.float32)],
    )(u, w2_w, w2_b.reshape(1, d))
    return g


# speedup vs baseline: 12.5566x; 1.0343x over previous
import functools

import jax
import jax.numpy as jnp
from jax import lax
from jax.experimental import pallas as pl
from jax.experimental.pallas import tpu as pltpu
from jax.experimental.pallas import tpu_sc as plsc

NC = 2
NS = 16
NW = NC * NS
C = 32


def _make_sc_scatter(n_agg, d, k_chunks):
    rows_per_sub = n_agg // NS
    kh = k_chunks // 8
    mesh = plsc.VectorSubcoreMesh(
        core_axis_name="c", subcore_axis_name="s", num_cores=NC, num_subcores=NS
    )

    @functools.partial(
        pl.kernel,
        out_type=jax.ShapeDtypeStruct((NC, n_agg, d), jnp.float32),
        mesh=mesh,
        scratch_types=[
            pltpu.VMEM((kh, C), jnp.int32),
            pltpu.VMEM((kh, C), jnp.int32),
            pltpu.VMEM((8, C, d), jnp.float32),
            pltpu.VMEM_SHARED((n_agg, d), jnp.float32),
        ] + [pltpu.SemaphoreType.DMA] * 10,
    )
    def sc_scatter(u_hbm, src_hbm, dst_hbm, zeros_hbm, out_hbm,
                   src_v, dst_v, rows8, agg_sh, *sems):
        c = lax.axis_index("c")
        s = lax.axis_index("s")
        wid = c * NS + s
        sub_rows = pl.ds(s * rows_per_sub, rows_per_sub)
        pltpu.async_copy(zeros_hbm.at[sub_rows], agg_sh.at[sub_rows], sems[8])
        barrier_done = False

        gsem = sems[:8]
        ssem = sems[8:]
        for h in range(8):
            pltpu.sync_copy(
                src_hbm.at[pl.ds(wid * k_chunks + h * kh, kh)], src_v)
            pltpu.sync_copy(
                dst_hbm.at[pl.ds(wid * k_chunks + h * kh, kh)], dst_v)

            n_octs = kh // 8
            for b in range(6):
                pltpu.async_copy(u_hbm.at[src_v.at[b]], rows8.at[b], gsem[b])
            if not barrier_done:
                pltpu.make_async_copy(zeros_hbm.at[sub_rows],
                                      agg_sh.at[sub_rows], sems[8]).wait()
                plsc.subcore_barrier()
                barrier_done = True

            def body(q, carry):
                for b in range(8):
                    j = 8 * q + b
                    buf = rows8.at[b]
                    pltpu.make_async_copy(u_hbm.at[src_v.at[j]], buf,
                                          gsem[b]).wait()

                    @pl.when(j >= 2)
                    def _():
                        pltpu.make_async_copy(
                            rows8.at[(b + 6) % 8],
                            agg_sh.at[dst_v.at[j - 2]], ssem[b % 2]).wait()

                    pltpu.async_copy(buf, agg_sh.at[dst_v.at[j]],
                                     ssem[b % 2], add=True)

                    @pl.when(j + 6 < kh)
                    def _():
                        pltpu.async_copy(u_hbm.at[src_v.at[j + 6]],
                                         rows8.at[(b + 6) % 8],
                                         gsem[(b + 6) % 8])
                return carry

            lax.fori_loop(0, n_octs, body, 0)
            pltpu.make_async_copy(rows8.at[(kh - 2) % 8],
                                  agg_sh.at[dst_v.at[kh - 2]], ssem[0]).wait()
            pltpu.make_async_copy(rows8.at[(kh - 1) % 8],
                                  agg_sh.at[dst_v.at[kh - 1]], ssem[1]).wait()

        plsc.subcore_barrier()
        pltpu.sync_copy(agg_sh.at[sub_rows], out_hbm.at[c, sub_rows])

    return sc_scatter


def _matmul_t(a, w):
    return lax.dot_general(a, w, (((1,), (1,)), ((), ())),
                           preferred_element_type=jnp.float32)


def _xlin_body(x_ref, w_ref, b_ref, out_ref):
    out_ref[...] = _matmul_t(x_ref[...], w_ref[...]) + b_ref[...]


def _mlp_body(agg0_ref, agg1_ref, xlin_ref, s1w, s1b, s2w, s2b, s3w, s3b,
              out_ref):
    a = agg0_ref[0] + agg1_ref[0]
    h = jax.nn.relu(_matmul_t(a, s1w[...]) + s1b[...])
    h = jax.nn.relu(_matmul_t(h, s2w[...]) + s2b[...])
    h = jnp.tanh(_matmul_t(h, s3w[...]) + s3b[...])
    out_ref[...] = jax.nn.relu(xlin_ref[...] + h)


def _mlp_readout_body(agg0_ref, agg1_ref, xlin_ref, s1w, s1b, s2w, s2b, s3w,
                      s3b, w2_ref, b2_ref, out_ref, acc_ref):
    i = pl.program_id(0)

    @pl.when(i == 0)
    def _():
        acc_ref[...] = jnp.zeros_like(acc_ref)

    a = agg0_ref[0] + agg1_ref[0]
    h = jax.nn.relu(_matmul_t(a, s1w[...]) + s1b[...])
    h = jax.nn.relu(_matmul_t(h, s2w[...]) + s2b[...])
    h = jnp.tanh(_matmul_t(h, s3w[...]) + s3b[...])
    u_blk = jax.nn.relu(xlin_ref[...] + h)
    acc_ref[...] += jnp.sum(u_blk, axis=0, keepdims=True)

    @pl.when(i == pl.num_programs(0) - 1)
    def _():
        out_ref[...] = _matmul_t(acc_ref[...], w2_ref[...]) + b2_ref[...]


def kernel(x, u, edge_index, w1_w, w1_b, s1_w, s1_b, s2_w, s2_b, s3_w, s3_b,
           w2_w, w2_b):
    n, d = u.shape
    t_rounds = 4

    n_agg = ((n + 8 * NS - 1) // (8 * NS)) * 8 * NS
    if n_agg == n:
        n_agg += 8 * NS
    e = edge_index.shape[1]
    k_chunks = ((e + NW * C - 1) // (NW * C) + 63) // 64 * 64
    e_pad = NW * k_chunks * C
    n_trash = n_agg - n

    pad = e_pad - e
    pad_ar = jnp.arange(pad, dtype=jnp.int32)
    src_p = jnp.concatenate([edge_index[0], pad_ar % n]).reshape(NW * k_chunks, C)
    dst_p = jnp.concatenate([edge_index[1], n + pad_ar % n_trash]).reshape(
        NW * k_chunks, C)
    zeros_hbm = jnp.zeros((n_agg, d), jnp.float32)

    sc_scatter = _make_sc_scatter(n_agg, d, k_chunks)

    bn = 2000
    grid = (n // bn,)
    w_spec = pl.BlockSpec((d, d), lambda i: (0, 0))
    b_spec = pl.BlockSpec((1, d), lambda i: (0, 0))
    row_spec = pl.BlockSpec((bn, d), lambda i: (i, 0))

    xlin = pl.pallas_call(
        _xlin_body,
        grid=grid,
        in_specs=[row_spec, w_spec, b_spec],
        out_specs=row_spec,
        out_shape=jax.ShapeDtypeStruct((n, d), jnp.float32),
    )(x, w1_w, w1_b.reshape(1, d))

    mlp = pl.pallas_call(
        _mlp_body,
        grid=grid,
        in_specs=[
            pl.BlockSpec((1, bn, d), lambda i: (0, i, 0)),
            pl.BlockSpec((1, bn, d), lambda i: (1, i, 0)),
            row_spec, w_spec, b_spec, w_spec, b_spec, w_spec, b_spec,
        ],
        out_specs=row_spec,
        out_shape=jax.ShapeDtypeStruct((n, d), jnp.float32),
    )

    mlp_readout = pl.pallas_call(
        _mlp_readout_body,
        grid=grid,
        in_specs=[
            pl.BlockSpec((1, bn, d), lambda i: (0, i, 0)),
            pl.BlockSpec((1, bn, d), lambda i: (1, i, 0)),
            row_spec, w_spec, b_spec, w_spec, b_spec, w_spec, b_spec,
            w_spec, b_spec,
        ],
        out_specs=pl.BlockSpec((1, d), lambda i: (0, 0)),
        out_shape=jax.ShapeDtypeStruct((1, d), jnp.float32),
        scratch_shapes=[pltpu.VMEM((1, d), jnp.float32)],
    )

    s1b2, s2b2, s3b2 = (b.reshape(1, d) for b in (s1_b, s2_b, s3_b))
    for t in range(t_rounds):
        agg2 = sc_scatter(u, src_p, dst_p, zeros_hbm)
        if t < t_rounds - 1:
            u = mlp(agg2, agg2, xlin, s1_w, s1b2, s2_w, s2b2, s3_w, s3b2)
        else:
            g = mlp_readout(agg2, agg2, xlin, s1_w, s1b2, s2_w, s2b2,
                            s3_w, s3b2, w2_w, w2_b.reshape(1, d))
    return g


# speedup vs baseline: 12.9856x; 1.0342x over previous
import functools

import jax
import jax.numpy as jnp
from jax import lax
from jax.experimental import pallas as pl
from jax.experimental.pallas import tpu as pltpu
from jax.experimental.pallas import tpu_sc as plsc

NC = 2
NS = 16
NW = NC * NS
C = 32


def _make_sc_scatter(n_agg, d, k_chunks):
    rows_per_sub = n_agg // NS
    kh = k_chunks // 8
    mesh = plsc.VectorSubcoreMesh(
        core_axis_name="c", subcore_axis_name="s", num_cores=NC, num_subcores=NS
    )

    @functools.partial(
        pl.kernel,
        out_type=jax.ShapeDtypeStruct((NC, n_agg, d), jnp.float32),
        mesh=mesh,
        scratch_types=[
            pltpu.VMEM((kh, C), jnp.int32),
            pltpu.VMEM((kh, C), jnp.int32),
            pltpu.VMEM((8, C, d), jnp.float32),
            pltpu.VMEM_SHARED((n_agg, d), jnp.float32),
        ] + [pltpu.SemaphoreType.DMA] * 10,
    )
    def sc_scatter(u_hbm, src_hbm, dst_hbm, zeros_hbm, out_hbm,
                   src_v, dst_v, rows8, agg_sh, *sems):
        c = lax.axis_index("c")
        s = lax.axis_index("s")
        wid = c * NS + s
        sub_rows = pl.ds(s * rows_per_sub, rows_per_sub)
        pltpu.async_copy(zeros_hbm.at[sub_rows], agg_sh.at[sub_rows], sems[8])

        gsem = sems[:8]
        ssem = sems[8:]
        n_octs = kh // 8
        pltpu.sync_copy(src_hbm.at[pl.ds(wid * k_chunks, kh)], src_v)
        for b in range(6):
            pltpu.async_copy(u_hbm.at[src_v.at[b]], rows8.at[b], gsem[b])
        pltpu.make_async_copy(zeros_hbm.at[sub_rows],
                              agg_sh.at[sub_rows], sems[8]).wait()
        plsc.subcore_barrier()

        for h in range(8):
            pltpu.sync_copy(
                dst_hbm.at[pl.ds(wid * k_chunks + h * kh, kh)], dst_v)

            def body(q, carry):
                for b in range(8):
                    j = 8 * q + b
                    buf = rows8.at[b]
                    pltpu.make_async_copy(u_hbm.at[src_v.at[j]], buf,
                                          gsem[b]).wait()

                    @pl.when(j >= 2)
                    def _():
                        pltpu.make_async_copy(
                            rows8.at[(b + 6) % 8],
                            agg_sh.at[dst_v.at[j - 2]], ssem[b % 2]).wait()

                    pltpu.async_copy(buf, agg_sh.at[dst_v.at[j]],
                                     ssem[b % 2], add=True)

                    @pl.when(j + 6 < kh)
                    def _():
                        pltpu.async_copy(u_hbm.at[src_v.at[j + 6]],
                                         rows8.at[(b + 6) % 8],
                                         gsem[(b + 6) % 8])
                return carry

            lax.fori_loop(0, n_octs, body, 0)
            if h < 7:
                pltpu.sync_copy(
                    src_hbm.at[pl.ds(wid * k_chunks + (h + 1) * kh, kh)],
                    src_v)
                for b in range(6):
                    pltpu.async_copy(u_hbm.at[src_v.at[b]], rows8.at[b],
                                     gsem[b])
            pltpu.make_async_copy(rows8.at[(kh - 2) % 8],
                                  agg_sh.at[dst_v.at[kh - 2]], ssem[0]).wait()
            pltpu.make_async_copy(rows8.at[(kh - 1) % 8],
                                  agg_sh.at[dst_v.at[kh - 1]], ssem[1]).wait()

        plsc.subcore_barrier()
        pltpu.sync_copy(agg_sh.at[sub_rows], out_hbm.at[c, sub_rows])

    return sc_scatter


def _matmul_t(a, w):
    return lax.dot_general(a, w, (((1,), (1,)), ((), ())),
                           preferred_element_type=jnp.float32)


def _xlin_body(x_ref, w_ref, b_ref, out_ref):
    out_ref[...] = _matmul_t(x_ref[...], w_ref[...]) + b_ref[...]


def _mlp_body(agg0_ref, agg1_ref, xlin_ref, s1w, s1b, s2w, s2b, s3w, s3b,
              out_ref):
    a = agg0_ref[0] + agg1_ref[0]
    h = jax.nn.relu(_matmul_t(a, s1w[...]) + s1b[...])
    h = jax.nn.relu(_matmul_t(h, s2w[...]) + s2b[...])
    h = jnp.tanh(_matmul_t(h, s3w[...]) + s3b[...])
    out_ref[...] = jax.nn.relu(xlin_ref[...] + h)


def _mlp_readout_body(agg0_ref, agg1_ref, xlin_ref, s1w, s1b, s2w, s2b, s3w,
                      s3b, w2_ref, b2_ref, out_ref, acc_ref):
    i = pl.program_id(0)

    @pl.when(i == 0)
    def _():
        acc_ref[...] = jnp.zeros_like(acc_ref)

    a = agg0_ref[0] + agg1_ref[0]
    h = jax.nn.relu(_matmul_t(a, s1w[...]) + s1b[...])
    h = jax.nn.relu(_matmul_t(h, s2w[...]) + s2b[...])
    h = jnp.tanh(_matmul_t(h, s3w[...]) + s3b[...])
    u_blk = jax.nn.relu(xlin_ref[...] + h)
    acc_ref[...] += jnp.sum(u_blk, axis=0, keepdims=True)

    @pl.when(i == pl.num_programs(0) - 1)
    def _():
        out_ref[...] = _matmul_t(acc_ref[...], w2_ref[...]) + b2_ref[...]


def kernel(x, u, edge_index, w1_w, w1_b, s1_w, s1_b, s2_w, s2_b, s3_w, s3_b,
           w2_w, w2_b):
    n, d = u.shape
    t_rounds = 4

    n_agg = ((n + 8 * NS - 1) // (8 * NS)) * 8 * NS
    if n_agg == n:
        n_agg += 8 * NS
    e = edge_index.shape[1]
    k_chunks = ((e + NW * C - 1) // (NW * C) + 63) // 64 * 64
    e_pad = NW * k_chunks * C
    n_trash = n_agg - n

    pad = e_pad - e
    pad_ar = jnp.arange(pad, dtype=jnp.int32)
    src_p = jnp.concatenate([edge_index[0], pad_ar % n]).reshape(NW * k_chunks, C)
    dst_p = jnp.concatenate([edge_index[1], n + pad_ar % n_trash]).reshape(
        NW * k_chunks, C)
    zeros_hbm = jnp.zeros((n_agg, d), jnp.float32)

    sc_scatter = _make_sc_scatter(n_agg, d, k_chunks)

    bn = 2000
    grid = (n // bn,)
    w_spec = pl.BlockSpec((d, d), lambda i: (0, 0))
    b_spec = pl.BlockSpec((1, d), lambda i: (0, 0))
    row_spec = pl.BlockSpec((bn, d), lambda i: (i, 0))

    xlin = pl.pallas_call(
        _xlin_body,
        grid=grid,
        in_specs=[row_spec, w_spec, b_spec],
        out_specs=row_spec,
        out_shape=jax.ShapeDtypeStruct((n, d), jnp.float32),
    )(x, w1_w, w1_b.reshape(1, d))

    mlp = pl.pallas_call(
        _mlp_body,
        grid=grid,
        in_specs=[
            pl.BlockSpec((1, bn, d), lambda i: (0, i, 0)),
            pl.BlockSpec((1, bn, d), lambda i: (1, i, 0)),
            row_spec, w_spec, b_spec, w_spec, b_spec, w_spec, b_spec,
        ],
        out_specs=row_spec,
        out_shape=jax.ShapeDtypeStruct((n, d), jnp.float32),
    )

    mlp_readout = pl.pallas_call(
        _mlp_readout_body,
        grid=grid,
        in_specs=[
            pl.BlockSpec((1, bn, d), lambda i: (0, i, 0)),
            pl.BlockSpec((1, bn, d), lambda i: (1, i, 0)),
            row_spec, w_spec, b_spec, w_spec, b_spec, w_spec, b_spec,
            w_spec, b_spec,
        ],
        out_specs=pl.BlockSpec((1, d), lambda i: (0, 0)),
        out_shape=jax.ShapeDtypeStruct((1, d), jnp.float32),
        scratch_shapes=[pltpu.VMEM((1, d), jnp.float32)],
    )

    s1b2, s2b2, s3b2 = (b.reshape(1, d) for b in (s1_b, s2_b, s3_b))
    for t in range(t_rounds):
        agg2 = sc_scatter(u, src_p, dst_p, zeros_hbm)
        if t < t_rounds - 1:
            u = mlp(agg2, agg2, xlin, s1_w, s1b2, s2_w, s2b2, s3_w, s3b2)
        else:
            g = mlp_readout(agg2, agg2, xlin, s1_w, s1b2, s2_w, s2b2,
                            s3_w, s3b2, w2_w, w2_b.reshape(1, d))
    return g
